# Initial kernel scaffold; baseline (speedup 1.0000x reference)
import jax
import jax.numpy as jnp
from jax.experimental import pallas as pl


def kernel(x, edge_type, edge_index, node_embed, Wout, bout, Wroot, W1, b1, prelu_a, W2, b2):
    raise NotImplementedError("write your pallas kernel here")


# speedup vs baseline: 7.3493x; 7.3493x over previous
import functools

import jax
import jax.numpy as jnp
from jax import lax
from jax.experimental import pallas as pl
from jax.experimental.pallas import tpu as pltpu
from jax.experimental.pallas import tpu_sc as plsc

N = 10000
E = 320000
H = 128
L = 3

NW = 32
B = 128
NCH = 79
E2 = NW * NCH * B
N1 = 10240
NPT = N1 // NW
NPS = N1 // 16
RB = 512
GRID = N1 // RB

_MESH = plsc.VectorSubcoreMesh(core_axis_name="c", subcore_axis_name="s")


def _zero_f32(ref, n):
    def body(i, _):
        ref[pl.ds(i * 16, 16)] = jnp.zeros((16,), jnp.float32)
        return 0
    lax.fori_loop(0, n // 16, body, 0)


@functools.partial(
    pl.kernel,
    out_type=(
        jax.ShapeDtypeStruct((N1, H), jnp.float32),
        jax.ShapeDtypeStruct((NW, NCH, B), jnp.int32),
        jax.ShapeDtypeStruct((2, N1), jnp.float32),
    ),
    mesh=_MESH,
    scratch_types=[
        pltpu.VMEM((4, 80), jnp.int32),
        pltpu.VMEM((NPT, H), jnp.float32),
        pltpu.VMEM((NCH, B), jnp.int32),
        pltpu.VMEM((NCH, B), jnp.int32),
        pltpu.VMEM((NCH, B), jnp.int32),
        pltpu.VMEM((B,), jnp.float32),
        pltpu.VMEM((NPS,), jnp.float32),
        pltpu.VMEM_SHARED((N1,), jnp.float32),
        pltpu.SemaphoreType.DMA,
    ],
)
def _sc_setup(x4, embed, row3, col3, z0, colp3, cnt_out,
              xv, zrows, rowv, colv, colpv, onesv, zerov, cnt_sp, sem):
    c = lax.axis_index("c")
    s = lax.axis_index("s")
    w = c * 16 + s

    pltpu.sync_copy(x4.at[w], xv)
    for j in range(4):
        pltpu.async_copy(embed.at[xv.at[j]], zrows.at[pl.ds(j * 80, 80)],
                         sem).wait()
    pltpu.sync_copy(zrows, z0.at[pl.ds(w * NPT, NPT)])

    pltpu.sync_copy(row3.at[w], rowv)
    pltpu.sync_copy(col3.at[w], colv)

    def cbody(j, _):
        for t in range(B // 16):
            r = rowv[j, pl.ds(t * 16, 16)]
            cc = colv[j, pl.ds(t * 16, 16)]
            colpv[j, pl.ds(t * 16, 16)] = jnp.where(
                r == cc, jnp.full((16,), N, jnp.int32), cc)
        return 0
    lax.fori_loop(0, NCH, cbody, 0)
    pltpu.sync_copy(colpv, colp3.at[w])

    _zero_f32(zerov, NPS)
    def obody(i, _):
        onesv[pl.ds(i * 16, 16)] = jnp.ones((16,), jnp.float32)
        return 0
    lax.fori_loop(0, B // 16, obody, 0)
    pltpu.sync_copy(zerov, cnt_sp.at[pl.ds(s * NPS, NPS)])
    plsc.subcore_barrier()

    def sbody(j, _):
        pltpu.sync_copy(onesv, cnt_sp.at[colpv.at[j]], add=True)
        return 0
    lax.fori_loop(0, NCH, sbody, 0)
    plsc.subcore_barrier()

    pltpu.sync_copy(cnt_sp.at[pl.ds(s * NPS, NPS)],
                    cnt_out.at[c, pl.ds(s * NPS, NPS)])


@functools.partial(
    pl.kernel,
    out_type=jax.ShapeDtypeStruct((2, N1, H), jnp.float32),
    mesh=_MESH,
    scratch_types=[
        pltpu.VMEM((NCH, B), jnp.int32),
        pltpu.VMEM((NCH, B), jnp.int32),
        pltpu.VMEM((B, H), jnp.float32),
        pltpu.VMEM_SHARED((N1, H), jnp.float32),
        pltpu.SemaphoreType.DMA,
    ],
)
def _sc_scatter(z, row3, colp3, s_out, rowv, colpv, rbuf, accum, sem):
    c = lax.axis_index("c")
    s = lax.axis_index("s")
    w = c * 16 + s

    pltpu.sync_copy(row3.at[w], rowv)
    pltpu.sync_copy(colp3.at[w], colpv)

    def zbody(i, _):
        for t in range(8):
            rbuf[i, pl.ds(t * 16, 16)] = jnp.zeros((16,), jnp.float32)
        return 0
    lax.fori_loop(0, B, zbody, 0)
    for k in range(NPS // B):
        pltpu.sync_copy(rbuf, accum.at[pl.ds(s * NPS + k * B, B)])
    plsc.subcore_barrier()

    def body(j, _):
        pltpu.async_copy(z.at[rowv.at[j]], rbuf, sem).wait()
        pltpu.sync_copy(rbuf, accum.at[colpv.at[j]], add=True)
        return 0
    lax.fori_loop(0, NCH, body, 0)
    plsc.subcore_barrier()

    for k in range(NPS // B):
        pltpu.sync_copy(accum.at[pl.ds(s * NPS + k * B, B)],
                        s_out.at[c, pl.ds(s * NPS + k * B, B)])


def _dense_body(s0, s1, zb, c0, c1, wout, wroot, bo, zout):
    di = 1.0 / (1.0 + c0[0, 0, :] + c1[0, 0, :])
    agg = (s0[...] + s1[...] + zb[...]) * di[:, None]
    acc = jnp.dot(agg, wout[...], preferred_element_type=jnp.float32)
    acc += jnp.dot(zb[...], wroot[...], preferred_element_type=jnp.float32)
    zout[...] = jnp.maximum(acc + bo[...], 0.0)


def _dense_head_body(s0, s1, zb, c0, c1, wout, wroot, bo,
                     w1, b1, pa, w2, b2, zout, pout):
    di = 1.0 / (1.0 + c0[0, 0, :] + c1[0, 0, :])
    agg = (s0[...] + s1[...] + zb[...]) * di[:, None]
    acc = jnp.dot(agg, wout[...], preferred_element_type=jnp.float32)
    acc += jnp.dot(zb[...], wroot[...], preferred_element_type=jnp.float32)
    zn = jnp.maximum(acc + bo[...], 0.0)
    zout[...] = zn
    h = jnp.dot(zn, w1[...], preferred_element_type=jnp.float32) + b1[...]
    h = jnp.where(h >= 0.0, h, pa[...] * h)
    pout[...] = jnp.dot(h, w2[...], preferred_element_type=jnp.float32) + b2[...]


_ROWS = pl.BlockSpec((RB, H), lambda i: (i, 0))
_CNT = pl.BlockSpec((1, 1, RB), lambda i: (i, 0, 0))
_WMAT = pl.BlockSpec((H, H), lambda i: (0, 0))
_BVEC = pl.BlockSpec((1, H), lambda i: (0, 0))

_dense = pl.pallas_call(
    _dense_body,
    grid=(GRID,),
    in_specs=[_ROWS, _ROWS, _ROWS, _CNT, _CNT, _WMAT, _WMAT, _BVEC],
    out_specs=_ROWS,
    out_shape=jax.ShapeDtypeStruct((N1, H), jnp.float32),
)

_dense_head = pl.pallas_call(
    _dense_head_body,
    grid=(GRID,),
    in_specs=[_ROWS, _ROWS, _ROWS, _CNT, _CNT, _WMAT, _WMAT, _BVEC,
              _WMAT, _BVEC, _BVEC, _WMAT, _BVEC],
    out_specs=(_ROWS, _ROWS),
    out_shape=(jax.ShapeDtypeStruct((N1, H), jnp.float32),
               jax.ShapeDtypeStruct((N1, H), jnp.float32)),
)


def kernel(x, edge_type, edge_index, node_embed, Wout, bout, Wroot,
           W1, b1, prelu_a, W2, b2):
    row = edge_index[0].astype(jnp.int32)
    col = edge_index[1].astype(jnp.int32)
    rowp = jnp.concatenate([row, jnp.zeros((E2 - E,), jnp.int32)])
    colp = jnp.concatenate([col, jnp.full((E2 - E,), N, jnp.int32)])
    row3 = rowp.reshape(NW, NCH, B)
    col3 = colp.reshape(NW, NCH, B)
    x4 = jnp.concatenate([x.astype(jnp.int32),
                          jnp.zeros((N1 - N,), jnp.int32)]).reshape(NW, 4, 80)

    z0, colp3, cnt = _sc_setup(x4, node_embed, row3, col3)
    c0 = cnt[0].reshape(GRID, 1, RB)
    c1 = cnt[1].reshape(GRID, 1, RB)
    pa = jnp.reshape(prelu_a, (1, 1)) * jnp.ones((1, H), jnp.float32)

    z = z0
    for l in range(L):
        S = _sc_scatter(z, row3, colp3)
        wl = Wout[l]
        rl = Wroot[l]
        bl = bout[l].reshape(1, H)
        if l < L - 1:
            z = _dense(S[0], S[1], z, c0, c1, wl, rl, bl)
        else:
            z, proj = _dense_head(S[0], S[1], z, c0, c1, wl, rl, bl,
                                  W1, b1.reshape(1, H), pa,
                                  W2, b2.reshape(1, H))
    return z[:N], proj[:N]
